# baseline (speedup 1.0000x reference)
import functools
import numpy as np
import jax
import jax.numpy as jnp
from jax import lax
from jax.experimental import pallas as pl
from jax.experimental.pallas import tpu as pltpu
from jax.experimental.pallas import tpu_sc as plsc

_B = 128
_DP = 32
_K = 32
_NW = 32
_B_SC = 32
_B_TC = _B - _B_SC
_CPAD_SC = 304
_NCV = _CPAD_SC // 16
_CPAD_TC = 384
_BIGC = 1 << 20


def _tables():
    tris = [(a, b, c)
            for a in range(_DP) for b in range(_DP) for c in range(_DP)
            if (a + 1) * (b + 1) * (c + 1) <= _K]
    t = np.array(tris, np.int32)
    return t, t.shape[0]


_TRIS, _C = _tables()

_TAB_SC = np.stack([
    np.concatenate([_TRIS[:, 0], np.full(_CPAD_SC - _C, _DP - 1, np.int32)]),
    np.concatenate([_TRIS[:, 1] + 32, np.full(_CPAD_SC - _C, 63, np.int32)]),
    np.concatenate([_TRIS[:, 2] + 64, np.full(_CPAD_SC - _C, 95, np.int32)]),
])

_TAB_TC = np.concatenate(
    [_TRIS, np.full((_CPAD_TC - _C, 3), _DP - 1, np.int32)], 0).T.copy()

_GDN = lax.GatherDimensionNumbers(
    offset_dims=(), collapsed_slice_dims=(0,), start_index_map=(0,))


def _lane_perm(v, idx):
    return lax.gather(v, idx[:, None], _GDN, slice_sizes=(1,),
                      mode=lax.GatherScatterMode.PROMISE_IN_BOUNDS)


def _lexmax(ak, av, bk, bv):
    c = (ak > bk) | ((ak == bk) & (av < bv))
    return (jnp.where(c, ak, bk), jnp.where(c, av, bv),
            jnp.where(c, bk, ak), jnp.where(c, bv, av))


def _merge32(x, y):
    xk0, xk1, xv0, xv1 = x
    yk0, yk1, yv0, yv1 = y
    ryk0, ryv0 = jnp.flip(yk1), jnp.flip(yv1)
    ryk1, ryv1 = jnp.flip(yk0), jnp.flip(yv0)
    z0k, z0v, _, _ = _lexmax(xk0, xv0, ryk0, ryv0)
    z1k, z1v, _, _ = _lexmax(xk1, xv1, ryk1, ryv1)
    uk, uv, vk, vv = _lexmax(z0k, z0v, z1k, z1v)
    s0k, s0v = plsc.sort_key_val(uk, uv, descending=True)
    s1k, s1v = plsc.sort_key_val(vk, vv, descending=True)
    return (s0k, s1k, s0v, s1v)


def _sc_call(z_sc, lt16, tab):
    mesh = plsc.VectorSubcoreMesh(core_axis_name="c", subcore_axis_name="s")

    @functools.partial(
        pl.kernel,
        out_type=[jax.ShapeDtypeStruct((_B_SC, _K), jnp.int32),
                  jax.ShapeDtypeStruct((_B_SC, _K), jnp.float32)],
        mesh=mesh,
        compiler_params=pltpu.CompilerParams(needs_layout_passes=False,
                                             skip_device_barrier=True),
        scratch_types=[
            pltpu.VMEM((1, 96), jnp.float32),
            pltpu.VMEM((96,), jnp.float32),
            pltpu.VMEM((96,), jnp.int32),
            pltpu.VMEM((3, _CPAD_SC), jnp.int32),
            pltpu.VMEM((16,), jnp.float32),
            pltpu.VMEM((1, _K), jnp.float32),
            pltpu.VMEM((1, _K), jnp.int32),
            pltpu.VMEM((32,), jnp.int32),
        ],
    )
    def sc(z_hbm, lt_hbm, tab_hbm, idx_hbm, w_hbm,
           zv, sv, av, tabv, ltv, wst, ist, flg):
        wid = lax.axis_index("s") * 2 + lax.axis_index("c")
        pltpu.sync_copy(z_hbm.at[pl.ds(wid, 1)], zv)
        pltpu.sync_copy(lt_hbm, ltv)
        pltpu.sync_copy(tab_hbm, tabv)
        iota = lax.broadcasted_iota(jnp.int32, (16,), 0)
        tau = jnp.exp(ltv[...])

        def fix32(o, k0, k1, v0, v1):
            snA = plsc.load_gather(sv, [iota + (o + 1)])
            snB = plsc.load_gather(sv, [jnp.minimum(iota + (o + 17), o + 31)])
            anA = plsc.load_gather(av, [iota + (o + 1)])
            anB = plsc.load_gather(av, [jnp.minimum(iota + (o + 17), o + 31)])
            fA = ((k0 == snA) & (v0 > anA)).astype(jnp.int32)
            fB = ((k1 == snB) & (v1 > anB) & (iota < 15)).astype(jnp.int32)
            flg[pl.ds(0, 16)] = fA
            flg[pl.ds(16, 16)] = fB
            fpA = jnp.where(iota > 0,
                            plsc.load_gather(flg, [jnp.maximum(iota - 1, 0)]),
                            0)
            fpB = plsc.load_gather(flg, [iota + 15])
            permA = iota + fA - fpA
            permB = iota + 16 + fB - fpB
            a0f = plsc.load_gather(av, [o + permA])
            a1f = plsc.load_gather(av, [o + permB])
            av[pl.ds(o, 16)] = a0f
            av[pl.ds(o + 16, 16)] = a1f
            return a0f, a1f

        for u in range(3):
            o = u * 32
            x0 = zv[0, pl.ds(o, 16)] / tau
            x1 = zv[0, pl.ds(o + 16, 16)] / tau
            m = jnp.maximum(x0, x1)
            for sh in (8, 4, 2, 1):
                m = jnp.maximum(m, _lane_perm(m, iota ^ sh))
            e0 = jnp.exp(x0 - m)
            e1 = jnp.exp(x1 - m)
            s = e0 + e1
            for sh in (8, 4, 2, 1):
                s = s + _lane_perm(s, iota ^ sh)
            p0 = e0 / s
            p1 = e1 / s
            k0, v0 = plsc.sort_key_val(p0, iota, descending=True)
            k1, v1 = plsc.sort_key_val(p1, iota + 16, descending=True)
            rbk = jnp.flip(k1)
            rbv = jnp.flip(v1)
            c0 = k0 >= rbk
            lok = jnp.where(c0, k0, rbk)
            lov = jnp.where(c0, v0, rbv)
            hik = jnp.where(c0, rbk, k0)
            hiv = jnp.where(c0, rbv, v0)
            s0k, s0v = plsc.sort_key_val(lok, lov, descending=True)
            s1k, s1v = plsc.sort_key_val(hik, hiv, descending=True)
            sv[pl.ds(o, 16)] = s0k
            sv[pl.ds(o + 16, 16)] = s1k
            av[pl.ds(o, 16)] = s0v
            av[pl.ds(o + 16, 16)] = s1v
            fix32(o, s0k, s1k, s0v, s1v)

        runs16 = []
        for ci in range(_NCV):
            o = ci * 16
            i0 = tabv[0, pl.ds(o, 16)]
            i1 = tabv[1, pl.ds(o, 16)]
            i2 = tabv[2, pl.ds(o, 16)]
            g0 = plsc.load_gather(sv, [i0])
            g1 = plsc.load_gather(sv, [i1])
            g2 = plsc.load_gather(sv, [i2])
            ga0 = plsc.load_gather(av, [i0])
            ga1 = plsc.load_gather(av, [i1])
            ga2 = plsc.load_gather(av, [i2])
            val = (g0 * g1) * g2
            comb = ga0 * (_DP * _DP) + ga1 * _DP + ga2
            if ci == _NCV - 1:
                pm = iota >= (_C - o)
                val = jnp.where(pm, -1.0, val)
                comb = jnp.where(pm, _BIGC, comb)
            runs16.append(plsc.sort_key_val(val, comb, descending=True))
        runs16.append((jnp.full((16,), -1.0, jnp.float32),
                       jnp.full((16,), _BIGC, jnp.int32)))

        runs32 = []
        for i in range(0, len(runs16), 2):
            ak, av_ = runs16[i]
            bk, bv = runs16[i + 1]
            rbk2, rbv2 = jnp.flip(bk), jnp.flip(bv)
            lok, lov, hik, hiv = _lexmax(ak, av_, rbk2, rbv2)
            s0k, s0v = plsc.sort_key_val(lok, lov, descending=True)
            s1k, s1v = plsc.sort_key_val(hik, hiv, descending=True)
            runs32.append((s0k, s1k, s0v, s1v))
        while len(runs32) > 1:
            nxt = [_merge32(runs32[i], runs32[i + 1])
                   for i in range(0, len(runs32) - 1, 2)]
            if len(runs32) % 2:
                nxt.append(runs32[-1])
            runs32 = nxt
        a0k, a1k, a0v, a1v = runs32[0]

        sv[pl.ds(0, 16)] = a0k
        sv[pl.ds(16, 16)] = a1k
        av[pl.ds(0, 16)] = a0v
        av[pl.ds(16, 16)] = a1v
        a0f, a1f = fix32(0, a0k, a1k, a0v, a1v)
        wst[0, pl.ds(0, 16)] = a0k
        wst[0, pl.ds(16, 16)] = a1k
        ist[0, pl.ds(0, 16)] = a0f
        ist[0, pl.ds(16, 16)] = a1f
        pltpu.sync_copy(wst, w_hbm.at[pl.ds(wid, 1)])
        pltpu.sync_copy(ist, idx_hbm.at[pl.ds(wid, 1)])

    return sc(z_sc, lt16, tab)


def _tc_body(z_ref, lt_ref, r0_ref, r1_ref, r2_ref, idx_ref, w_ref):
    n = _B_TC
    tau = jnp.exp(lt_ref[0])
    z = z_ref[:, :]
    lane32 = lax.broadcasted_iota(jnp.int32, (n, _DP), 1)
    ilane = lax.broadcasted_iota(jnp.int32, (n, _DP, _DP), 1)
    jlane = lax.broadcasted_iota(jnp.int32, (n, _DP, _DP), 2)

    probs = []
    ranks = []
    for u in range(3):
        x = z[:, u * _DP:(u + 1) * _DP] / tau
        mx = jnp.max(x, axis=1, keepdims=True)
        e = jnp.exp(x - mx)
        sm = jnp.sum(e, axis=1, keepdims=True)
        p = e / sm
        pi = p[:, :, None]
        pj = p[:, None, :]
        beats = (pj > pi) | ((pj == pi) & (jlane < ilane))
        r = jnp.sum(beats.astype(jnp.int32), axis=2)
        probs.append(p)
        ranks.append(r)

    vs = []
    cs = []
    for u, r_ref in enumerate((r0_ref, r1_ref, r2_ref)):
        rt = r_ref[:, :]
        vu = jnp.zeros((n, _CPAD_TC), jnp.float32)
        iu = jnp.zeros((n, _CPAD_TC), jnp.int32)
        for i in range(_DP):
            msk = rt == ranks[u][:, i:i + 1]
            vu = jnp.where(msk, probs[u][:, i:i + 1], vu)
            iu = jnp.where(msk, i, iu)
        vs.append(vu)
        cs.append(iu)

    cand_v = (vs[0] * vs[1]) * vs[2]
    comb = cs[0] * (_DP * _DP) + cs[1] * _DP + cs[2]
    clane = lax.broadcasted_iota(jnp.int32, (n, _CPAD_TC), 1)
    padm = clane >= _C
    cand_v = jnp.where(padm, -1.0, cand_v)
    comb = jnp.where(padm, _BIGC, comb)

    w_out = jnp.zeros((n, _K), jnp.float32)
    i_out = jnp.zeros((n, _K), jnp.int32)
    for t in range(_K):
        mv = jnp.max(cand_v, axis=1, keepdims=True)
        bi = jnp.min(jnp.where(cand_v == mv, comb, _BIGC),
                     axis=1, keepdims=True)
        w_out = jnp.where(lane32 == t, mv, w_out)
        i_out = jnp.where(lane32 == t, bi, i_out)
        cand_v = jnp.where((cand_v == mv) & (comb == bi), -1.0, cand_v)

    idx_ref[:, :] = i_out
    w_ref[:, :] = w_out


def _tc_call(z_tc, log_tau):
    r0 = jnp.asarray(_TAB_TC[0].reshape(1, _CPAD_TC))
    r1 = jnp.asarray(_TAB_TC[1].reshape(1, _CPAD_TC))
    r2 = jnp.asarray(_TAB_TC[2].reshape(1, _CPAD_TC))
    return pl.pallas_call(
        _tc_body,
        out_shape=[
            jax.ShapeDtypeStruct((_B_TC, _K), jnp.int32),
            jax.ShapeDtypeStruct((_B_TC, _K), jnp.float32),
        ],
        in_specs=[
            pl.BlockSpec(memory_space=pltpu.VMEM),
            pl.BlockSpec(memory_space=pltpu.SMEM),
            pl.BlockSpec(memory_space=pltpu.VMEM),
            pl.BlockSpec(memory_space=pltpu.VMEM),
            pl.BlockSpec(memory_space=pltpu.VMEM),
        ],
        out_specs=[
            pl.BlockSpec(memory_space=pltpu.VMEM),
            pl.BlockSpec(memory_space=pltpu.VMEM),
        ],
    )(z_tc, log_tau, r0, r1, r2)


def kernel(z, log_tau):
    lt16 = jnp.broadcast_to(log_tau, (16,))
    tab_sc = jnp.asarray(_TAB_SC)
    sc_idx, sc_w = _sc_call(z[_B_TC:], lt16, tab_sc)
    tc_idx, tc_w = _tc_call(z[:_B_TC], log_tau)
    indices = jnp.concatenate([tc_idx, sc_idx], axis=0)
    weights = jnp.concatenate([tc_w, sc_w], axis=0)
    return (indices, weights)


# speedup vs baseline: 1.3805x; 1.3805x over previous
import functools
import numpy as np
import jax
import jax.numpy as jnp
from jax import lax
from jax.experimental import pallas as pl
from jax.experimental.pallas import tpu as pltpu
from jax.experimental.pallas import tpu_sc as plsc

_B = 128
_DP = 32
_K = 32
_NW = 32
_RPW = _B // _NW
_CPAD = 304
_NCV = _CPAD // 16
_BIGC = 1 << 20


def _tables():
    tris = [(a, b, c)
            for a in range(_DP) for b in range(_DP) for c in range(_DP)
            if (a + 1) * (b + 1) * (c + 1) <= _K]
    t = np.array(tris, np.int32)
    c = t.shape[0]
    t = np.concatenate([t, np.full((_CPAD - c, 3), _DP - 1, np.int32)], 0)
    flat = np.stack([t[:, 0], t[:, 1] + 32, t[:, 2] + 64])
    return flat, c


_TAB, _C = _tables()

_GDN = lax.GatherDimensionNumbers(
    offset_dims=(), collapsed_slice_dims=(0,), start_index_map=(0,))


def _lane_perm(v, idx):
    return lax.gather(v, idx[:, None], _GDN, slice_sizes=(1,),
                      mode=lax.GatherScatterMode.PROMISE_IN_BOUNDS)


def _lexmax(ak, av, bk, bv):
    c = (ak > bk) | ((ak == bk) & (av < bv))
    return (jnp.where(c, ak, bk), jnp.where(c, av, bv),
            jnp.where(c, bk, ak), jnp.where(c, bv, av))


def _merge32(x, y):
    xk0, xk1, xv0, xv1 = x
    yk0, yk1, yv0, yv1 = y
    ryk0, ryv0 = jnp.flip(yk1), jnp.flip(yv1)
    ryk1, ryv1 = jnp.flip(yk0), jnp.flip(yv0)
    z0k, z0v, _, _ = _lexmax(xk0, xv0, ryk0, ryv0)
    z1k, z1v, _, _ = _lexmax(xk1, xv1, ryk1, ryv1)
    uk, uv, vk, vv = _lexmax(z0k, z0v, z1k, z1v)
    s0k, s0v = plsc.sort_key_val(uk, uv, descending=True)
    s1k, s1v = plsc.sort_key_val(vk, vv, descending=True)
    return (s0k, s1k, s0v, s1v)


def kernel(z, log_tau):
    lt16 = jnp.broadcast_to(log_tau, (16,))
    tab = jnp.asarray(_TAB)
    mesh = plsc.VectorSubcoreMesh(core_axis_name="c", subcore_axis_name="s")

    @functools.partial(
        pl.kernel,
        out_type=[jax.ShapeDtypeStruct((_B, _K), jnp.int32),
                  jax.ShapeDtypeStruct((_B, _K), jnp.float32)],
        mesh=mesh,
        compiler_params=pltpu.CompilerParams(needs_layout_passes=False,
                                             skip_device_barrier=True),
        scratch_types=[
            pltpu.VMEM((_RPW, 96), jnp.float32),
            pltpu.VMEM((96,), jnp.float32),
            pltpu.VMEM((96,), jnp.int32),
            pltpu.VMEM((3, _CPAD), jnp.int32),
            pltpu.VMEM((16,), jnp.float32),
            pltpu.VMEM((_RPW, _K), jnp.float32),
            pltpu.VMEM((_RPW, _K), jnp.int32),
            pltpu.VMEM((32,), jnp.int32),
            pltpu.SemaphoreType.DMA,
            pltpu.SemaphoreType.DMA,
            pltpu.SemaphoreType.DMA,
        ],
    )
    def sc(z_hbm, lt_hbm, tab_hbm, idx_hbm, w_hbm,
           zv, sv, av, tabv, ltv, wst, ist, flg, sem_z, sem_lt, sem_tab):
        wid = lax.axis_index("s") * 2 + lax.axis_index("c")
        base = wid * _RPW
        cp_z = pltpu.make_async_copy(z_hbm.at[pl.ds(base, _RPW)], zv, sem_z)
        cp_lt = pltpu.make_async_copy(lt_hbm, ltv, sem_lt)
        cp_tab = pltpu.make_async_copy(tab_hbm, tabv, sem_tab)
        cp_z.start()
        cp_lt.start()
        cp_tab.start()
        cp_z.wait()
        cp_lt.wait()
        iota = lax.broadcasted_iota(jnp.int32, (16,), 0)
        tau = jnp.exp(ltv[...])
        tab_pending = [True]

        def wait_tab():
            if tab_pending:
                cp_tab.wait()
                tab_pending.clear()

        def fix32(o, k0, k1, v0, v1):
            snA = plsc.load_gather(sv, [iota + (o + 1)])
            snB = plsc.load_gather(sv, [jnp.minimum(iota + (o + 17), o + 31)])
            anA = plsc.load_gather(av, [iota + (o + 1)])
            anB = plsc.load_gather(av, [jnp.minimum(iota + (o + 17), o + 31)])
            fA = ((k0 == snA) & (v0 > anA)).astype(jnp.int32)
            fB = ((k1 == snB) & (v1 > anB) & (iota < 15)).astype(jnp.int32)
            flg[pl.ds(0, 16)] = fA
            flg[pl.ds(16, 16)] = fB
            fpA = jnp.where(iota > 0,
                            plsc.load_gather(flg, [jnp.maximum(iota - 1, 0)]),
                            0)
            fpB = plsc.load_gather(flg, [iota + 15])
            permA = iota + fA - fpA
            permB = iota + 16 + fB - fpB
            a0f = plsc.load_gather(av, [o + permA])
            a1f = plsc.load_gather(av, [o + permB])
            av[pl.ds(o, 16)] = a0f
            av[pl.ds(o + 16, 16)] = a1f
            return a0f, a1f

        for r in range(_RPW):
            for u in range(3):
                o = u * 32
                x0 = zv[r, pl.ds(o, 16)] / tau
                x1 = zv[r, pl.ds(o + 16, 16)] / tau
                m = jnp.maximum(x0, x1)
                for sh in (8, 4, 2, 1):
                    m = jnp.maximum(m, _lane_perm(m, iota ^ sh))
                e0 = jnp.exp(x0 - m)
                e1 = jnp.exp(x1 - m)
                s = e0 + e1
                for sh in (8, 4, 2, 1):
                    s = s + _lane_perm(s, iota ^ sh)
                p0 = e0 / s
                p1 = e1 / s
                k0, v0 = plsc.sort_key_val(p0, iota, descending=True)
                k1, v1 = plsc.sort_key_val(p1, iota + 16, descending=True)
                rbk = jnp.flip(k1)
                rbv = jnp.flip(v1)
                c0 = k0 >= rbk
                lok = jnp.where(c0, k0, rbk)
                lov = jnp.where(c0, v0, rbv)
                hik = jnp.where(c0, rbk, k0)
                hiv = jnp.where(c0, rbv, v0)
                s0k, s0v = plsc.sort_key_val(lok, lov, descending=True)
                s1k, s1v = plsc.sort_key_val(hik, hiv, descending=True)
                sv[pl.ds(o, 16)] = s0k
                sv[pl.ds(o + 16, 16)] = s1k
                av[pl.ds(o, 16)] = s0v
                av[pl.ds(o + 16, 16)] = s1v
                fix32(o, s0k, s1k, s0v, s1v)

            wait_tab()
            runs16 = []
            for ci in range(_NCV):
                o = ci * 16
                i0 = tabv[0, pl.ds(o, 16)]
                i1 = tabv[1, pl.ds(o, 16)]
                i2 = tabv[2, pl.ds(o, 16)]
                g0 = plsc.load_gather(sv, [i0])
                g1 = plsc.load_gather(sv, [i1])
                g2 = plsc.load_gather(sv, [i2])
                ga0 = plsc.load_gather(av, [i0])
                ga1 = plsc.load_gather(av, [i1])
                ga2 = plsc.load_gather(av, [i2])
                val = (g0 * g1) * g2
                comb = ga0 * (_DP * _DP) + ga1 * _DP + ga2
                if ci == _NCV - 1:
                    pm = iota >= (_C - o)
                    val = jnp.where(pm, -1.0, val)
                    comb = jnp.where(pm, _BIGC, comb)
                runs16.append(plsc.sort_key_val(val, comb, descending=True))
            runs16.append((jnp.full((16,), -1.0, jnp.float32),
                           jnp.full((16,), _BIGC, jnp.int32)))

            runs32 = []
            for i in range(0, len(runs16), 2):
                ak, av_ = runs16[i]
                bk, bv = runs16[i + 1]
                rbk2, rbv2 = jnp.flip(bk), jnp.flip(bv)
                lok, lov, hik, hiv = _lexmax(ak, av_, rbk2, rbv2)
                s0k, s0v = plsc.sort_key_val(lok, lov, descending=True)
                s1k, s1v = plsc.sort_key_val(hik, hiv, descending=True)
                runs32.append((s0k, s1k, s0v, s1v))
            while len(runs32) > 1:
                nxt = [_merge32(runs32[i], runs32[i + 1])
                       for i in range(0, len(runs32) - 1, 2)]
                if len(runs32) % 2:
                    nxt.append(runs32[-1])
                runs32 = nxt
            a0k, a1k, a0v, a1v = runs32[0]

            sv[pl.ds(0, 16)] = a0k
            sv[pl.ds(16, 16)] = a1k
            av[pl.ds(0, 16)] = a0v
            av[pl.ds(16, 16)] = a1v
            a0f, a1f = fix32(0, a0k, a1k, a0v, a1v)
            wst[r, pl.ds(0, 16)] = a0k
            wst[r, pl.ds(16, 16)] = a1k
            ist[r, pl.ds(0, 16)] = a0f
            ist[r, pl.ds(16, 16)] = a1f

        pltpu.sync_copy(wst, w_hbm.at[pl.ds(base, _RPW)])
        pltpu.sync_copy(ist, idx_hbm.at[pl.ds(base, _RPW)])

    indices, weights = sc(z, lt16, tab)
    return (indices, weights)


# speedup vs baseline: 1.4025x; 1.0160x over previous
import functools
import numpy as np
import jax
import jax.numpy as jnp
from jax import lax
from jax.experimental import pallas as pl
from jax.experimental.pallas import tpu as pltpu
from jax.experimental.pallas import tpu_sc as plsc

_B = 128
_DP = 32
_K = 32
_NW = 32
_RPW = _B // _NW
_CPAD = 304
_NCV = _CPAD // 16
_BIGC = 1 << 20


def _tables():
    tris = [(a, b, c)
            for a in range(_DP) for b in range(_DP) for c in range(_DP)
            if (a + 1) * (b + 1) * (c + 1) <= _K]
    t = np.array(tris, np.int32)
    c = t.shape[0]
    t = np.concatenate([t, np.full((_CPAD - c, 3), _DP - 1, np.int32)], 0)
    flat = np.stack([t[:, 0], t[:, 1] + 32, t[:, 2] + 64])
    return flat, c


_TAB, _C = _tables()

_GDN = lax.GatherDimensionNumbers(
    offset_dims=(), collapsed_slice_dims=(0,), start_index_map=(0,))


def _lane_perm(v, idx):
    return lax.gather(v, idx[:, None], _GDN, slice_sizes=(1,),
                      mode=lax.GatherScatterMode.PROMISE_IN_BOUNDS)


def _lexmax(ak, av, bk, bv):
    c = (ak > bk) | ((ak == bk) & (av < bv))
    return (jnp.where(c, ak, bk), jnp.where(c, av, bv),
            jnp.where(c, bk, ak), jnp.where(c, bv, av))


def _merge32(x, y):
    xk0, xk1, xv0, xv1 = x
    yk0, yk1, yv0, yv1 = y
    ryk0, ryv0 = jnp.flip(yk1), jnp.flip(yv1)
    ryk1, ryv1 = jnp.flip(yk0), jnp.flip(yv0)
    z0k, z0v, _, _ = _lexmax(xk0, xv0, ryk0, ryv0)
    z1k, z1v, _, _ = _lexmax(xk1, xv1, ryk1, ryv1)
    uk, uv, vk, vv = _lexmax(z0k, z0v, z1k, z1v)
    s0k, s0v = plsc.sort_key_val(uk, uv, descending=True)
    s1k, s1v = plsc.sort_key_val(vk, vv, descending=True)
    return (s0k, s1k, s0v, s1v)


def kernel(z, log_tau):
    lt16 = jnp.broadcast_to(log_tau, (16,))
    tab = jnp.asarray(_TAB)
    mesh = plsc.VectorSubcoreMesh(core_axis_name="c", subcore_axis_name="s")

    @functools.partial(
        pl.kernel,
        out_type=[jax.ShapeDtypeStruct((_B, _K), jnp.int32),
                  jax.ShapeDtypeStruct((_B, _K), jnp.float32)],
        mesh=mesh,
        compiler_params=pltpu.CompilerParams(needs_layout_passes=False,
                                             skip_device_barrier=True),
        scratch_types=[
            pltpu.VMEM((_RPW, 96), jnp.float32),
            pltpu.VMEM((96,), jnp.float32),
            pltpu.VMEM((96,), jnp.int32),
            pltpu.VMEM((3, _CPAD), jnp.int32),
            pltpu.VMEM((16,), jnp.float32),
            pltpu.VMEM((_RPW, _K), jnp.float32),
            pltpu.VMEM((_RPW, _K), jnp.int32),
            pltpu.SemaphoreType.DMA,
            pltpu.SemaphoreType.DMA,
            pltpu.SemaphoreType.DMA,
        ],
    )
    def sc(z_hbm, lt_hbm, tab_hbm, idx_hbm, w_hbm,
           zv, sv, av, tabv, ltv, wst, ist, sem_z, sem_lt, sem_tab):
        wid = lax.axis_index("s") * 2 + lax.axis_index("c")
        base = wid * _RPW
        cp_z = pltpu.make_async_copy(z_hbm.at[pl.ds(base, _RPW)], zv, sem_z)
        cp_lt = pltpu.make_async_copy(lt_hbm, ltv, sem_lt)
        cp_tab = pltpu.make_async_copy(tab_hbm, tabv, sem_tab)
        cp_z.start()
        cp_lt.start()
        cp_tab.start()
        cp_z.wait()
        cp_lt.wait()
        iota = lax.broadcasted_iota(jnp.int32, (16,), 0)
        tau = jnp.exp(ltv[...])
        tab_pending = [True]

        def wait_tab():
            if tab_pending:
                cp_tab.wait()
                tab_pending.clear()

        def fix32(k0, k1, v0, v1):
            up = jnp.minimum(iota + 1, 15)
            dn = jnp.maximum(iota - 1, 0)
            zero16 = iota * 0
            sn0 = jnp.where(iota == 15, _lane_perm(k1, zero16),
                            _lane_perm(k0, up))
            an0 = jnp.where(iota == 15, _lane_perm(v1, zero16),
                            _lane_perm(v0, up))
            sn1 = _lane_perm(k1, up)
            an1 = _lane_perm(v1, up)
            f0 = ((k0 == sn0) & (v0 > an0)).astype(jnp.int32)
            f1 = ((k1 == sn1) & (v1 > an1) & (iota < 15)).astype(jnp.int32)
            fp0 = jnp.where(iota == 0, 0, _lane_perm(f0, dn))
            fp1 = jnp.where(iota == 0, _lane_perm(f0, zero16 + 15),
                            _lane_perm(f1, dn))
            ap0 = _lane_perm(v0, dn)
            ap1 = jnp.where(iota == 0, _lane_perm(v0, zero16 + 15),
                            _lane_perm(v1, dn))
            a0f = jnp.where(f0 > 0, an0, jnp.where(fp0 > 0, ap0, v0))
            a1f = jnp.where(f1 > 0, an1, jnp.where(fp1 > 0, ap1, v1))
            return a0f, a1f

        for r in range(_RPW):
            for u in range(3):
                o = u * 32
                x0 = zv[r, pl.ds(o, 16)] / tau
                x1 = zv[r, pl.ds(o + 16, 16)] / tau
                m = jnp.maximum(x0, x1)
                for sh in (8, 4, 2, 1):
                    m = jnp.maximum(m, _lane_perm(m, iota ^ sh))
                e0 = jnp.exp(x0 - m)
                e1 = jnp.exp(x1 - m)
                s = e0 + e1
                for sh in (8, 4, 2, 1):
                    s = s + _lane_perm(s, iota ^ sh)
                p0 = e0 / s
                p1 = e1 / s
                k0, v0 = plsc.sort_key_val(p0, iota, descending=True)
                k1, v1 = plsc.sort_key_val(p1, iota + 16, descending=True)
                rbk = jnp.flip(k1)
                rbv = jnp.flip(v1)
                c0 = k0 >= rbk
                lok = jnp.where(c0, k0, rbk)
                lov = jnp.where(c0, v0, rbv)
                hik = jnp.where(c0, rbk, k0)
                hiv = jnp.where(c0, rbv, v0)
                s0k, s0v = plsc.sort_key_val(lok, lov, descending=True)
                s1k, s1v = plsc.sort_key_val(hik, hiv, descending=True)
                s0v, s1v = fix32(s0k, s1k, s0v, s1v)
                sv[pl.ds(o, 16)] = s0k
                sv[pl.ds(o + 16, 16)] = s1k
                av[pl.ds(o, 16)] = s0v
                av[pl.ds(o + 16, 16)] = s1v

            wait_tab()
            gath = []
            for ci in range(_NCV):
                o = ci * 16
                i0 = tabv[0, pl.ds(o, 16)]
                i1 = tabv[1, pl.ds(o, 16)]
                i2 = tabv[2, pl.ds(o, 16)]
                g0 = plsc.load_gather(sv, [i0])
                g1 = plsc.load_gather(sv, [i1])
                g2 = plsc.load_gather(sv, [i2])
                ga0 = plsc.load_gather(av, [i0])
                ga1 = plsc.load_gather(av, [i1])
                ga2 = plsc.load_gather(av, [i2])
                val = (g0 * g1) * g2
                comb = ga0 * (_DP * _DP) + ga1 * _DP + ga2
                if ci == _NCV - 1:
                    pm = iota >= (_C - o)
                    val = jnp.where(pm, -1.0, val)
                    comb = jnp.where(pm, _BIGC, comb)
                gath.append((val, comb))

            runs32 = [(gath[0][0], gath[1][0], gath[0][1], gath[1][1])]
            runs16 = [gath[2]]
            for val, comb in gath[3:]:
                runs16.append(plsc.sort_key_val(val, comb, descending=True))
            for i in range(0, len(runs16) - 1, 2):
                ak, av_ = runs16[i]
                bk, bv = runs16[i + 1]
                rbk2, rbv2 = jnp.flip(bk), jnp.flip(bv)
                lok, lov, hik, hiv = _lexmax(ak, av_, rbk2, rbv2)
                s0k, s0v = plsc.sort_key_val(lok, lov, descending=True)
                s1k, s1v = plsc.sort_key_val(hik, hiv, descending=True)
                runs32.append((s0k, s1k, s0v, s1v))
            lk, lv = runs16[-1]
            runs32.append((lk, jnp.full((16,), -1.0, jnp.float32),
                           lv, jnp.full((16,), _BIGC, jnp.int32)))
            while len(runs32) > 1:
                nxt = [_merge32(runs32[i], runs32[i + 1])
                       for i in range(0, len(runs32) - 1, 2)]
                if len(runs32) % 2:
                    nxt.append(runs32[-1])
                runs32 = nxt
            a0k, a1k, a0v, a1v = runs32[0]

            a0f, a1f = fix32(a0k, a1k, a0v, a1v)
            wst[r, pl.ds(0, 16)] = a0k
            wst[r, pl.ds(16, 16)] = a1k
            ist[r, pl.ds(0, 16)] = a0f
            ist[r, pl.ds(16, 16)] = a1f

        pltpu.sync_copy(wst, w_hbm.at[pl.ds(base, _RPW)])
        pltpu.sync_copy(ist, idx_hbm.at[pl.ds(base, _RPW)])

    indices, weights = sc(z, lt16, tab)
    return (indices, weights)


# speedup vs baseline: 1.4284x; 1.0184x over previous
import functools
import numpy as np
import jax
import jax.numpy as jnp
from jax import lax
from jax.experimental import pallas as pl
from jax.experimental.pallas import tpu as pltpu
from jax.experimental.pallas import tpu_sc as plsc

_B = 128
_DP = 32
_K = 32
_NW = 32
_RPW = _B // _NW
_CPAD = 304
_NCV = _CPAD // 16
_BIGC = 1 << 20


def _tables():
    tris = [(a, b, c)
            for a in range(_DP) for b in range(_DP) for c in range(_DP)
            if (a + 1) * (b + 1) * (c + 1) <= _K]
    t = np.array(tris, np.int32)
    c = t.shape[0]
    t = np.concatenate([t, np.full((_CPAD - c, 3), _DP - 1, np.int32)], 0)
    flat = np.stack([t[:, 0], t[:, 1] + 32, t[:, 2] + 64])
    return flat, c


_TAB, _C = _tables()

_GDN = lax.GatherDimensionNumbers(
    offset_dims=(), collapsed_slice_dims=(0,), start_index_map=(0,))


def _lane_perm(v, idx):
    return lax.gather(v, idx[:, None], _GDN, slice_sizes=(1,),
                      mode=lax.GatherScatterMode.PROMISE_IN_BOUNDS)


def _lexmax(ak, av, bk, bv):
    c = (ak > bk) | ((ak == bk) & (av < bv))
    return (jnp.where(c, ak, bk), jnp.where(c, av, bv),
            jnp.where(c, bk, ak), jnp.where(c, bv, av))


def _merge32(x, y):
    xk0, xk1, xv0, xv1 = x
    yk0, yk1, yv0, yv1 = y
    ryk0, ryv0 = jnp.flip(yk1), jnp.flip(yv1)
    ryk1, ryv1 = jnp.flip(yk0), jnp.flip(yv0)
    z0k, z0v, _, _ = _lexmax(xk0, xv0, ryk0, ryv0)
    z1k, z1v, _, _ = _lexmax(xk1, xv1, ryk1, ryv1)
    uk, uv, vk, vv = _lexmax(z0k, z0v, z1k, z1v)
    s0k, s0v = plsc.sort_key_val(uk, uv, descending=True)
    s1k, s1v = plsc.sort_key_val(vk, vv, descending=True)
    return (s0k, s1k, s0v, s1v)


def kernel(z, log_tau):
    lt16 = jnp.broadcast_to(log_tau, (16,))
    tab = jnp.asarray(_TAB)
    mesh = plsc.VectorSubcoreMesh(core_axis_name="c", subcore_axis_name="s")

    @functools.partial(
        pl.kernel,
        out_type=[jax.ShapeDtypeStruct((_B, _K), jnp.int32),
                  jax.ShapeDtypeStruct((_B, _K), jnp.float32)],
        mesh=mesh,
        compiler_params=pltpu.CompilerParams(needs_layout_passes=False,
                                             skip_device_barrier=True),
        scratch_types=[
            pltpu.VMEM((_RPW, 96), jnp.float32),
            pltpu.VMEM((96,), jnp.float32),
            pltpu.VMEM((96,), jnp.int32),
            pltpu.VMEM((3, _CPAD), jnp.int32),
            pltpu.VMEM((16,), jnp.float32),
            pltpu.VMEM((_RPW, _K), jnp.float32),
            pltpu.VMEM((_RPW, _K), jnp.int32),
            pltpu.SemaphoreType.DMA,
            pltpu.SemaphoreType.DMA,
            pltpu.SemaphoreType.DMA,
        ],
    )
    def sc(z_hbm, lt_hbm, tab_hbm, idx_hbm, w_hbm,
           zv, sv, av, tabv, ltv, wst, ist, sem_z, sem_lt, sem_tab):
        wid = lax.axis_index("s") * 2 + lax.axis_index("c")
        base = wid * _RPW
        cp_z = pltpu.make_async_copy(z_hbm.at[pl.ds(base, _RPW)], zv, sem_z)
        cp_lt = pltpu.make_async_copy(lt_hbm, ltv, sem_lt)
        cp_tab = pltpu.make_async_copy(tab_hbm, tabv, sem_tab)
        cp_z.start()
        cp_lt.start()
        cp_tab.start()
        cp_z.wait()
        cp_lt.wait()
        iota = lax.broadcasted_iota(jnp.int32, (16,), 0)
        tau = jnp.exp(ltv[...])
        tab_pending = [True]

        def wait_tab():
            if tab_pending:
                cp_tab.wait()
                tab_pending.clear()

        def fix32(k0, k1, v0, v1):
            up = jnp.minimum(iota + 1, 15)
            dn = jnp.maximum(iota - 1, 0)
            zero16 = iota * 0
            sn0 = jnp.where(iota == 15, _lane_perm(k1, zero16),
                            _lane_perm(k0, up))
            an0 = jnp.where(iota == 15, _lane_perm(v1, zero16),
                            _lane_perm(v0, up))
            sn1 = _lane_perm(k1, up)
            an1 = _lane_perm(v1, up)
            f0 = ((k0 == sn0) & (v0 > an0)).astype(jnp.int32)
            f1 = ((k1 == sn1) & (v1 > an1) & (iota < 15)).astype(jnp.int32)
            fp0 = jnp.where(iota == 0, 0, _lane_perm(f0, dn))
            fp1 = jnp.where(iota == 0, _lane_perm(f0, zero16 + 15),
                            _lane_perm(f1, dn))
            ap0 = _lane_perm(v0, dn)
            ap1 = jnp.where(iota == 0, _lane_perm(v0, zero16 + 15),
                            _lane_perm(v1, dn))
            a0f = jnp.where(f0 > 0, an0, jnp.where(fp0 > 0, ap0, v0))
            a1f = jnp.where(f1 > 0, an1, jnp.where(fp1 > 0, ap1, v1))
            return a0f, a1f

        for r in range(_RPW):
            fk = {}
            for u in range(3):
                o = u * 32
                x0 = zv[r, pl.ds(o, 16)] / tau
                x1 = zv[r, pl.ds(o + 16, 16)] / tau
                m = jnp.maximum(x0, x1)
                for sh in (8, 4, 2, 1):
                    m = jnp.maximum(m, _lane_perm(m, iota ^ sh))
                e0 = jnp.exp(x0 - m)
                e1 = jnp.exp(x1 - m)
                s = e0 + e1
                for sh in (8, 4, 2, 1):
                    s = s + _lane_perm(s, iota ^ sh)
                p0 = e0 / s
                p1 = e1 / s
                k0, v0 = plsc.sort_key_val(p0, iota, descending=True)
                k1, v1 = plsc.sort_key_val(p1, iota + 16, descending=True)
                rbk = jnp.flip(k1)
                rbv = jnp.flip(v1)
                c0 = k0 >= rbk
                lok = jnp.where(c0, k0, rbk)
                lov = jnp.where(c0, v0, rbv)
                hik = jnp.where(c0, rbk, k0)
                hiv = jnp.where(c0, rbv, v0)
                s0k, s0v = plsc.sort_key_val(lok, lov, descending=True)
                s1k, s1v = plsc.sort_key_val(hik, hiv, descending=True)
                s0v, s1v = fix32(s0k, s1k, s0v, s1v)
                sv[pl.ds(o, 16)] = s0k
                sv[pl.ds(o + 16, 16)] = s1k
                av[pl.ds(o, 16)] = s0v
                av[pl.ds(o + 16, 16)] = s1v
                fk[u] = (s0k, s1k, s0v, s1v)

            zero16 = iota * 0
            b0 = _lane_perm(fk[0][0], zero16)
            c0 = _lane_perm(fk[0][2], zero16) * (_DP * _DP)
            b1 = _lane_perm(fk[1][0], zero16)
            c1 = _lane_perm(fk[1][2], zero16) * _DP
            b1b = _lane_perm(fk[1][0], zero16 + 1)
            c1b = _lane_perm(fk[1][2], zero16 + 1) * _DP
            k2_0, k2_1, a2_0, a2_1 = fk[2]
            b01 = b0 * b1
            b01b = b0 * b1b
            val0 = b01 * k2_0
            val1 = b01 * k2_1
            val2 = b01b * k2_0
            comb0 = c0 + c1 + a2_0
            comb1 = c0 + c1 + a2_1
            comb2 = c0 + c1b + a2_0

            wait_tab()
            gath = []
            for ci in range(3, _NCV):
                o = ci * 16
                i0 = tabv[0, pl.ds(o, 16)]
                i1 = tabv[1, pl.ds(o, 16)]
                i2 = tabv[2, pl.ds(o, 16)]
                g0 = plsc.load_gather(sv, [i0])
                g1 = plsc.load_gather(sv, [i1])
                g2 = plsc.load_gather(sv, [i2])
                ga0 = plsc.load_gather(av, [i0])
                ga1 = plsc.load_gather(av, [i1])
                ga2 = plsc.load_gather(av, [i2])
                val = (g0 * g1) * g2
                comb = ga0 * (_DP * _DP) + ga1 * _DP + ga2
                if ci == _NCV - 1:
                    pm = iota >= (_C - o)
                    val = jnp.where(pm, -1.0, val)
                    comb = jnp.where(pm, _BIGC, comb)
                gath.append((val, comb))

            runs32 = [(val0, val1, comb0, comb1)]
            runs16 = [(val2, comb2)]
            for val, comb in gath:
                runs16.append(plsc.sort_key_val(val, comb, descending=True))
            for i in range(0, len(runs16) - 1, 2):
                ak, av_ = runs16[i]
                bk, bv = runs16[i + 1]
                rbk2, rbv2 = jnp.flip(bk), jnp.flip(bv)
                lok, lov, hik, hiv = _lexmax(ak, av_, rbk2, rbv2)
                s0k, s0v = plsc.sort_key_val(lok, lov, descending=True)
                s1k, s1v = plsc.sort_key_val(hik, hiv, descending=True)
                runs32.append((s0k, s1k, s0v, s1v))
            lk, lv = runs16[-1]
            runs32.append((lk, jnp.full((16,), -1.0, jnp.float32),
                           lv, jnp.full((16,), _BIGC, jnp.int32)))
            while len(runs32) > 1:
                nxt = [_merge32(runs32[i], runs32[i + 1])
                       for i in range(0, len(runs32) - 1, 2)]
                if len(runs32) % 2:
                    nxt.append(runs32[-1])
                runs32 = nxt
            a0k, a1k, a0v, a1v = runs32[0]

            a0f, a1f = fix32(a0k, a1k, a0v, a1v)
            wst[r, pl.ds(0, 16)] = a0k
            wst[r, pl.ds(16, 16)] = a1k
            ist[r, pl.ds(0, 16)] = a0f
            ist[r, pl.ds(16, 16)] = a1f

        pltpu.sync_copy(wst, w_hbm.at[pl.ds(base, _RPW)])
        pltpu.sync_copy(ist, idx_hbm.at[pl.ds(base, _RPW)])

    indices, weights = sc(z, lt16, tab)
    return (indices, weights)
